# baseline (speedup 1.0000x reference)
import functools

import jax
import jax.numpy as jnp
from jax import lax
from jax.experimental import pallas as pl
from jax.experimental.pallas import tpu as pltpu
from jax.experimental.pallas import tpu_sc as plsc

NV = 50000
NE = 50000
E = 800000
VDIM = 128
HID = 64
HEADS = 4
F = HEADS * HID

L = 16
NC = 2
NS = 16
NW = NC * NS

SB1 = 640
NSB1 = E // SB1
SB1_BIG = NSB1 // NW + 1
SB1_SMALL = NSB1 // NW
N_BIG1 = NSB1 - SB1_SMALL * NW

NCH = 25
CH = NE // NCH
SB2 = 2000
PER_SUB2 = E // NS
NSB2 = PER_SUB2 // SB2
FL = 128

_iota16 = None


def _enc_body(x_ref, w1_ref, b1_ref, g_ref, be_ref, w2_ref, b2_ref, o_ref,
              obf_ref=None):
    x = x_ref[...]
    h = jnp.maximum(jnp.dot(x, w1_ref[...], preferred_element_type=jnp.float32)
                    + b1_ref[...][None, :], 0.0)
    mu = jnp.mean(h, axis=1, keepdims=True)
    var = jnp.mean((h - mu) * (h - mu), axis=1, keepdims=True)
    y = (h - mu) * lax.rsqrt(var + 1e-5) * g_ref[...][None, :] + be_ref[...][None, :]
    f = jnp.dot(y, w2_ref[...], preferred_element_type=jnp.float32) \
        + b2_ref[...][None, :]
    o_ref[...] = f
    if obf_ref is not None:
        obf_ref[...] = f.astype(jnp.bfloat16)


def _encode(x, w1, b1, g, be, w2, b2, want_bf16=False):
    B = 2000
    n = x.shape[0]
    grid = n // B
    if want_bf16:
        out_specs = (pl.BlockSpec((B, F), lambda i: (i, 0)),
                     pl.BlockSpec((B, F), lambda i: (i, 0)))
        out_shape = (jax.ShapeDtypeStruct((n, F), jnp.float32),
                     jax.ShapeDtypeStruct((n, F), jnp.bfloat16))
        body = _enc_body
    else:
        out_specs = pl.BlockSpec((B, F), lambda i: (i, 0))
        out_shape = jax.ShapeDtypeStruct((n, F), jnp.float32)

        def body(x_ref, w1_ref, b1_ref, g_ref, be_ref, w2_ref, b2_ref, o_ref):
            _enc_body(x_ref, w1_ref, b1_ref, g_ref, be_ref, w2_ref, b2_ref,
                      o_ref)
    return pl.pallas_call(
        body,
        grid=(grid,),
        in_specs=[
            pl.BlockSpec((B, VDIM), lambda i: (i, 0)),
            pl.BlockSpec((VDIM, HID), lambda i: (0, 0)),
            pl.BlockSpec((HID,), lambda i: (0,)),
            pl.BlockSpec((HID,), lambda i: (0,)),
            pl.BlockSpec((HID,), lambda i: (0,)),
            pl.BlockSpec((HID, F), lambda i: (0, 0)),
            pl.BlockSpec((F,), lambda i: (0,)),
        ],
        out_specs=out_specs,
        out_shape=out_shape,
    )(x, w1, b1, g, be, w2, b2)


def _fin_body(num_ref, den_ref, cb_ref, wfp_ref, bfp_ref, o_ref):
    den = den_ref[0] + den_ref[1]
    num = num_ref[...]
    acc = jnp.zeros((num.shape[0], HID), jnp.float32)
    for h in range(HEADS):
        dh = den[:, h:h + 1]
        dh = jnp.where(dh > 0.0, dh, 1.0)
        acc = acc + num[:, h * HID:(h + 1) * HID] / dh \
            + cb_ref[...][None, h * HID:(h + 1) * HID]
    eo = acc * (1.0 / HEADS)
    o_ref[...] = jnp.maximum(
        jnp.dot(eo, wfp_ref[...], preferred_element_type=jnp.float32)
        + bfp_ref[...][None, :], 0.0)


def _finalize(num, den2, conv_bias, w_fp, b_fp):
    B = 2000
    grid = NE // B
    return pl.pallas_call(
        _fin_body,
        grid=(grid,),
        in_specs=[
            pl.BlockSpec((B, F), lambda i: (i, 0)),
            pl.BlockSpec((2, B, 16), lambda i: (0, i, 0)),
            pl.BlockSpec((F,), lambda i: (0,)),
            pl.BlockSpec((HID, HID), lambda i: (0, 0)),
            pl.BlockSpec((HID,), lambda i: (0,)),
        ],
        out_specs=pl.BlockSpec((B, HID), lambda i: (i, 0)),
        out_shape=jax.ShapeDtypeStruct((NE, HID), jnp.float32),
    )(num, den2, conv_bias, w_fp, b_fp)


def _p1_body(fsrc_hbm, fdst_hbm, src_hbm, dst_hbm, attn_hbm, rec_hbm, den_hbm,
             attn_v, src_stage, dst_stage, fs_buf, fd_buf, rec_stage,
             den_stage, idx_buf, zden, shared_den, sem):
    c = lax.axis_index("c")
    s = lax.axis_index("s")
    widx = s * NC + c
    iota = jnp.arange(L, dtype=jnp.int32)

    pltpu.sync_copy(attn_hbm, attn_v)

    def _zrow(r, _):
        den_stage[r, :] = jnp.zeros((16,), jnp.float32)
        return 0
    lax.fori_loop(0, 128, _zrow, 0)

    def _zr(r, _):
        zden[r, :] = jnp.zeros((16,), jnp.float32)
        return 0
    lax.fori_loop(0, 400, _zr, 0)
    for q in range(8):
        ch = s + q * NS

        @pl.when(ch < 125)
        def _():
            pltpu.sync_copy(zden, shared_den.at[pl.ds(ch * 400, 400)])
    plsc.subcore_barrier()

    nsb = jnp.where(widx < N_BIG1, SB1_BIG, SB1_SMALL)
    sb0 = jnp.where(widx < N_BIG1, widx * SB1_BIG,
                    N_BIG1 * SB1_BIG + (widx - N_BIG1) * SB1_SMALL)

    row_off = iota * F

    NBATCH = SB1 // L

    def _issue(b, slot):
        src16 = src_stage[pl.ds(b * L, L)]
        dst16 = dst_stage[pl.ds(b * L, L)]
        pltpu.async_copy(fsrc_hbm.at[src16],
                         fs_buf.at[pl.ds(slot * L, L)], sem)
        pltpu.async_copy(fdst_hbm.at[dst16],
                         fd_buf.at[pl.ds(slot * L, L)], sem)

    def _drain(slot):
        pltpu.make_async_copy(fsrc_hbm.at[pl.ds(0, L)],
                              fs_buf.at[pl.ds(slot * L, L)], sem).wait()
        pltpu.make_async_copy(fdst_hbm.at[pl.ds(0, L)],
                              fd_buf.at[pl.ds(slot * L, L)], sem).wait()

    def _subblock(k, _):
        base = (sb0 + k) * SB1
        pltpu.sync_copy(src_hbm.at[pl.ds(base, SB1)], src_stage)
        pltpu.sync_copy(dst_hbm.at[pl.ds(base, SB1)], dst_stage)
        _issue(jnp.int32(0), jnp.int32(0))

        def _batch(b, _):
            b2 = lax.rem(b, 8)
            slot = lax.rem(b, 2)
            _drain(slot)

            @pl.when(b + 1 < NBATCH)
            def _():
                _issue(b + 1, 1 - slot)
            src16 = src_stage[pl.ds(b * L, L)]
            dst16 = dst_stage[pl.ds(b * L, L)]
            rows16 = slot * L + iota
            accs = []
            for h in range(HEADS):
                acc = jnp.zeros((L,), jnp.float32)
                for jj in range(HID // L):
                    av = attn_v[h, pl.ds(jj * L, L)]
                    base_d = h * HID + jj * L
                    for k in range(L):
                        rot = (iota + k) % L
                        cols = base_d + rot
                        a = plsc.load_gather(fs_buf, [rows16, cols])
                        bdd = plsc.load_gather(fd_buf, [rows16, cols])
                        z = a + bdd
                        z = jnp.where(z >= 0.0, z, 0.2 * z)
                        acc = acc + av[rot] * z
                accs.append(acc)
            rows = b * L + iota
            drows = b2 * L + iota
            for h in range(HEADS):
                ex = jnp.exp(jnp.clip(accs[h], -60.0, 60.0))
                plsc.store_scatter(rec_stage,
                                   [rows, jnp.full((L,), h, jnp.int32)], ex)
                plsc.store_scatter(den_stage,
                                   [drows, jnp.full((L,), h, jnp.int32)], ex)
            idx_buf[0, pl.ds(b2 * L, L)] = dst16

            @pl.when(b2 == 7)
            def _():
                pltpu.sync_copy(den_stage, shared_den.at[idx_buf.at[0]],
                                add=True)
            return 0
        lax.fori_loop(0, NBATCH, _batch, 0)
        pltpu.sync_copy(rec_stage, rec_hbm.at[pl.ds(base, SB1)])
        return 0

    lax.fori_loop(0, nsb, _subblock, 0)

    plsc.subcore_barrier()
    for q in range(8):
        ch = s + q * NS

        @pl.when(ch < 125)
        def _():
            pltpu.sync_copy(shared_den.at[pl.ds(ch * 400, 400)],
                            den_hbm.at[c, pl.ds(ch * 400, 400)])


def _pass1(f_src, f_dst, src_ids, dst_ids, attn):
    mesh = plsc.VectorSubcoreMesh(core_axis_name="c", subcore_axis_name="s")
    kfn = pl.kernel(
        _p1_body,
        compiler_params=pltpu.CompilerParams(use_tc_tiling_on_sc=False, needs_layout_passes=False),
        out_type=(jax.ShapeDtypeStruct((E, 4), jnp.float32),
                  jax.ShapeDtypeStruct((2, NE, 16), jnp.float32)),
        mesh=mesh,
        scratch_types=[
            pltpu.VMEM((HEADS, HID), jnp.float32),
            pltpu.VMEM((SB1,), jnp.int32),
            pltpu.VMEM((SB1,), jnp.int32),
            pltpu.VMEM((2 * L, F), jnp.float32),
            pltpu.VMEM((2 * L, F), jnp.float32),
            pltpu.VMEM((SB1, 4), jnp.float32),
            pltpu.VMEM((128, 16), jnp.float32),
            pltpu.VMEM((1, 128), jnp.int32),
            pltpu.VMEM((400, 16), jnp.float32),
            pltpu.VMEM_SHARED((NE, 16), jnp.float32),
            pltpu.SemaphoreType.DMA,
        ],
    )
    return kfn(f_src, f_dst, src_ids, dst_ids, attn)


def _p2_flush(n, fcnt, rec_hbm, fsrc_hbm, eid_buf, drel_buf, scomp_buf,
              gid_buf, drl_buf, src_buf, rec_v, contrib, shared_num,
              sem, sem2, iota):
    fslot = lax.rem(fcnt, 2)

    @pl.when(fcnt >= 2)
    def _():
        pltpu.make_async_copy(contrib.at[fslot],
                              shared_num.at[drl_buf.at[fslot]], sem2).wait()
    for sb in range(FL // L):
        m = sb * L + iota < n
        ev = eid_buf[pl.ds(sb * L, L)]
        gid_buf[0, pl.ds(sb * L, L)] = jnp.where(m, ev, 0)
        dv = drel_buf[pl.ds(sb * L, L)]
        drl_buf[fslot, pl.ds(sb * L, L)] = jnp.where(m, dv, 0)
        sv = scomp_buf[pl.ds(sb * L, L)]
        src_buf[0, pl.ds(sb * L, L)] = jnp.where(m, sv, 0)
    cp1 = pltpu.async_copy(rec_hbm.at[gid_buf.at[0]], rec_v, sem)
    cp2 = pltpu.async_copy(fsrc_hbm.at[src_buf.at[0]], contrib.at[fslot],
                           sem)
    cp1.wait()
    cp2.wait()

    def _group16(g, _):
        rows = g * L + iota
        m = rows < n
        ex_vs = [jnp.where(m, plsc.load_gather(
            rec_v, [rows, jnp.full((L,), h, jnp.int32)]), 0.0)
            for h in range(HEADS)]
        for r2 in range(L):
            r = g * L + r2
            for h in range(HEADS):
                w = ex_vs[h][r2]
                for j in range(HID // L):
                    d0 = h * HID + j * L
                    contrib[fslot, r, pl.ds(d0, L)] = \
                        w * contrib[fslot, r, pl.ds(d0, L)]
        return 0
    lax.fori_loop(0, FL // L, _group16, 0)
    pltpu.async_copy(contrib.at[fslot], shared_num.at[drl_buf.at[fslot]],
                     sem2, add=True)


def _p2_body(rec_hbm, fsrc_hbm, src_hbm, dst_hbm, num_hbm,
             dst_stage, src_stage, eid_buf, drel_buf, scomp_buf, gid_buf,
             drl_buf, src_buf, rec_v, contrib, zbuf, shared_num,
             sem, sem2):
    c = lax.axis_index("c")
    s = lax.axis_index("s")
    iota = jnp.arange(L, dtype=jnp.int32)

    def _zr(r, _):
        for j in range(F // L):
            zbuf[r, pl.ds(j * L, L)] = jnp.zeros((L,), jnp.float32)
        return 0
    lax.fori_loop(0, 16, _zr, 0)

    e0 = s * PER_SUB2
    r0 = s * 128

    def _chunk(i, _):
        chunk = 2 * i + c
        lo = chunk * CH

        @pl.when(chunk < NCH)
        def _chunk_body():
            _chunk_inner(chunk, lo)
        return 0

    def _chunk_inner(chunk, lo):
        @pl.when(s < NS - 1)
        def _():
            cps = [pltpu.async_copy(
                zbuf, shared_num.at[pl.ds(r0 + 16 * q, 16)], sem)
                for q in range(8)]
            for cp in cps:
                cp.wait()

        @pl.when(s == NS - 1)
        def _():
            cps = [pltpu.async_copy(
                zbuf, shared_num.at[pl.ds(1920 + 16 * q, 16)], sem)
                for q in range(5)]
            for cp in cps:
                cp.wait()
        plsc.subcore_barrier()

        def _sissue(k, slot):
            base = e0 + k * SB2
            pltpu.async_copy(dst_hbm.at[pl.ds(base, SB2)],
                             dst_stage.at[slot], sem)
            pltpu.async_copy(src_hbm.at[pl.ds(base, SB2)],
                             src_stage.at[slot], sem)

        def _sdrain(slot):
            pltpu.make_async_copy(dst_hbm.at[pl.ds(0, SB2)],
                                  dst_stage.at[slot], sem).wait()
            pltpu.make_async_copy(src_hbm.at[pl.ds(0, SB2)],
                                  src_stage.at[slot], sem).wait()

        _sissue(jnp.int32(0), jnp.int32(0))

        def _scan_sub(k, carry):
            base = e0 + k * SB2
            slot = lax.rem(k, 2)
            _sdrain(slot)

            @pl.when(k + 1 < NSB2)
            def _():
                _sissue(k + 1, 1 - slot)

            def _scan_batch(b, carry):
                cnt, fcnt = carry
                d16 = dst_stage[slot, pl.ds(b * L, L)]
                s16 = src_stage[slot, pl.ds(b * L, L)]
                drel = d16 - lo
                m = (drel >= 0) & (drel < CH)
                eidv = base + b * L + iota
                plsc.store_compressed(eid_buf.at[pl.ds(cnt, L)], eidv,
                                      mask=m)
                plsc.store_compressed(drel_buf.at[pl.ds(cnt, L)], drel,
                                      mask=m)
                plsc.store_compressed(scomp_buf.at[pl.ds(cnt, L)], s16,
                                      mask=m)
                cnt = cnt + jnp.sum(m.astype(jnp.int32))

                def _do_flush(carry):
                    cnt, fcnt = carry
                    _p2_flush(jnp.int32(FL), fcnt, rec_hbm, fsrc_hbm,
                              eid_buf, drel_buf, scomp_buf, gid_buf, drl_buf,
                              src_buf, rec_v, contrib, shared_num,
                              sem, sem2, iota)
                    eid_buf[pl.ds(0, L)] = eid_buf[pl.ds(FL, L)]
                    drel_buf[pl.ds(0, L)] = drel_buf[pl.ds(FL, L)]
                    scomp_buf[pl.ds(0, L)] = scomp_buf[pl.ds(FL, L)]
                    return (cnt - FL, fcnt + 1)

                return lax.cond(cnt >= FL, _do_flush, lambda x: x,
                                (cnt, fcnt))

            return lax.fori_loop(0, SB2 // L, _scan_batch, carry)

        cnt, fcnt = lax.fori_loop(0, NSB2, _scan_sub,
                                  (jnp.int32(0), jnp.int32(0)))

        @pl.when(cnt > 0)
        def _():
            _p2_flush(cnt, fcnt, rec_hbm, fsrc_hbm, eid_buf, drel_buf,
                      scomp_buf, gid_buf, drl_buf, src_buf, rec_v,
                      contrib, shared_num, sem, sem2, iota)
        fcnt = fcnt + jnp.where(cnt > 0, 1, 0)

        @pl.when(fcnt >= 1)
        def _():
            pltpu.make_async_copy(
                contrib.at[lax.rem(fcnt - 1, 2)],
                shared_num.at[drl_buf.at[lax.rem(fcnt - 1, 2)]], sem2).wait()

        @pl.when(fcnt >= 2)
        def _():
            pltpu.make_async_copy(
                contrib.at[lax.rem(fcnt, 2)],
                shared_num.at[drl_buf.at[lax.rem(fcnt, 2)]], sem2).wait()

        plsc.subcore_barrier()

        @pl.when(s < NS - 1)
        def _():
            pltpu.sync_copy(shared_num.at[pl.ds(r0, 128)],
                            num_hbm.at[pl.ds(lo + r0, 128)])

        @pl.when(s == NS - 1)
        def _():
            pltpu.sync_copy(shared_num.at[pl.ds(1920, 80)],
                            num_hbm.at[pl.ds(lo + 1920, 80)])

    lax.fori_loop(0, (NCH + 1) // 2, _chunk, 0)


def _pass2(rec, f_src, src_ids, dst_ids):
    mesh = plsc.VectorSubcoreMesh(core_axis_name="c", subcore_axis_name="s")
    kfn = pl.kernel(
        _p2_body,
        compiler_params=pltpu.CompilerParams(use_tc_tiling_on_sc=False, needs_layout_passes=False),
        out_type=jax.ShapeDtypeStruct((NE, F), jnp.float32),
        mesh=mesh,
        scratch_types=[
            pltpu.VMEM((2, SB2), jnp.int32),
            pltpu.VMEM((2, SB2), jnp.int32),
            pltpu.VMEM((FL + 32,), jnp.int32),
            pltpu.VMEM((FL + 32,), jnp.int32),
            pltpu.VMEM((FL + 32,), jnp.int32),
            pltpu.VMEM((1, FL), jnp.int32),
            pltpu.VMEM((2, FL), jnp.int32),
            pltpu.VMEM((1, FL), jnp.int32),
            pltpu.VMEM((FL, 4), jnp.float32),
            pltpu.VMEM((2, FL, F), jnp.float32),
            pltpu.VMEM((16, F), jnp.float32),
            pltpu.VMEM_SHARED((CH, F), jnp.float32),
            pltpu.SemaphoreType.DMA,
            pltpu.SemaphoreType.DMA,
        ],
    )
    return kfn(rec, f_src, src_ids, dst_ids)


def kernel(vehicle_features, edge_node_features, edge_index,
           w_ve, b_ve, g_ve, be_ve,
           w_ee, b_ee, g_ee, be_ee,
           w_src, b_src, w_dst, b_dst, attn, conv_bias,
           w_fp, b_fp):
    f_src = _encode(vehicle_features, w_ve, b_ve, g_ve, be_ve, w_src, b_src)
    f_dst = _encode(edge_node_features, w_ee, b_ee, g_ee, be_ee, w_dst, b_dst)
    src_ids = edge_index[0]
    dst_ids = edge_index[1]
    rec, den2 = _pass1(f_src, f_dst, src_ids, dst_ids, attn)
    num = _pass2(rec, f_src, src_ids, dst_ids)
    return _finalize(num, den2, conv_bias, w_fp, b_fp)


# speedup vs baseline: 1.0003x; 1.0003x over previous
import functools

import jax
import jax.numpy as jnp
from jax import lax
from jax.experimental import pallas as pl
from jax.experimental.pallas import tpu as pltpu
from jax.experimental.pallas import tpu_sc as plsc

NV = 50000
NE = 50000
E = 800000
VDIM = 128
HID = 64
HEADS = 4
F = HEADS * HID

L = 16
NC = 2
NS = 16
NW = NC * NS

SB1 = 640
NSB1 = E // SB1
SB1_BIG = NSB1 // NW + 1
SB1_SMALL = NSB1 // NW
N_BIG1 = NSB1 - SB1_SMALL * NW

NCH = 25
CH = NE // NCH
SB2 = 2000
PER_SUB2 = E // NS
NSB2 = PER_SUB2 // SB2
FL = 128

_iota16 = None


def _enc_body(x_ref, w1_ref, b1_ref, g_ref, be_ref, w2_ref, b2_ref, o_ref,
              obf_ref=None):
    x = x_ref[...]
    h = jnp.maximum(jnp.dot(x, w1_ref[...], preferred_element_type=jnp.float32)
                    + b1_ref[...][None, :], 0.0)
    mu = jnp.mean(h, axis=1, keepdims=True)
    var = jnp.mean((h - mu) * (h - mu), axis=1, keepdims=True)
    y = (h - mu) * lax.rsqrt(var + 1e-5) * g_ref[...][None, :] + be_ref[...][None, :]
    f = jnp.dot(y, w2_ref[...], preferred_element_type=jnp.float32) \
        + b2_ref[...][None, :]
    o_ref[...] = f
    if obf_ref is not None:
        obf_ref[...] = f.astype(jnp.bfloat16)


def _encode(x, w1, b1, g, be, w2, b2, want_bf16=False):
    B = 2000
    n = x.shape[0]
    grid = n // B
    if want_bf16:
        out_specs = (pl.BlockSpec((B, F), lambda i: (i, 0)),
                     pl.BlockSpec((B, F), lambda i: (i, 0)))
        out_shape = (jax.ShapeDtypeStruct((n, F), jnp.float32),
                     jax.ShapeDtypeStruct((n, F), jnp.bfloat16))
        body = _enc_body
    else:
        out_specs = pl.BlockSpec((B, F), lambda i: (i, 0))
        out_shape = jax.ShapeDtypeStruct((n, F), jnp.float32)

        def body(x_ref, w1_ref, b1_ref, g_ref, be_ref, w2_ref, b2_ref, o_ref):
            _enc_body(x_ref, w1_ref, b1_ref, g_ref, be_ref, w2_ref, b2_ref,
                      o_ref)
    return pl.pallas_call(
        body,
        grid=(grid,),
        in_specs=[
            pl.BlockSpec((B, VDIM), lambda i: (i, 0)),
            pl.BlockSpec((VDIM, HID), lambda i: (0, 0)),
            pl.BlockSpec((HID,), lambda i: (0,)),
            pl.BlockSpec((HID,), lambda i: (0,)),
            pl.BlockSpec((HID,), lambda i: (0,)),
            pl.BlockSpec((HID, F), lambda i: (0, 0)),
            pl.BlockSpec((F,), lambda i: (0,)),
        ],
        out_specs=out_specs,
        out_shape=out_shape,
    )(x, w1, b1, g, be, w2, b2)


def _fin_body(num_ref, den_ref, cb_ref, wfp_ref, bfp_ref, o_ref):
    den = den_ref[0] + den_ref[1]
    num = num_ref[...]
    acc = jnp.zeros((num.shape[0], HID), jnp.float32)
    for h in range(HEADS):
        dh = den[:, h:h + 1]
        dh = jnp.where(dh > 0.0, dh, 1.0)
        acc = acc + num[:, h * HID:(h + 1) * HID] / dh \
            + cb_ref[...][None, h * HID:(h + 1) * HID]
    eo = acc * (1.0 / HEADS)
    o_ref[...] = jnp.maximum(
        jnp.dot(eo, wfp_ref[...], preferred_element_type=jnp.float32)
        + bfp_ref[...][None, :], 0.0)


def _finalize(num, den2, conv_bias, w_fp, b_fp):
    B = 2000
    grid = NE // B
    return pl.pallas_call(
        _fin_body,
        grid=(grid,),
        in_specs=[
            pl.BlockSpec((B, F), lambda i: (i, 0)),
            pl.BlockSpec((2, B, 16), lambda i: (0, i, 0)),
            pl.BlockSpec((F,), lambda i: (0,)),
            pl.BlockSpec((HID, HID), lambda i: (0, 0)),
            pl.BlockSpec((HID,), lambda i: (0,)),
        ],
        out_specs=pl.BlockSpec((B, HID), lambda i: (i, 0)),
        out_shape=jax.ShapeDtypeStruct((NE, HID), jnp.float32),
    )(num, den2, conv_bias, w_fp, b_fp)


def _p1_body(fsrc_hbm, fdst_hbm, src_hbm, dst_hbm, attn_hbm, rec_hbm, den_hbm,
             attn_v, src_stage, dst_stage, fs_buf, fd_buf, rec_stage,
             den_stage, idx_buf, zden, shared_den, sem):
    c = lax.axis_index("c")
    s = lax.axis_index("s")
    widx = s * NC + c
    iota = jnp.arange(L, dtype=jnp.int32)

    pltpu.sync_copy(attn_hbm, attn_v)

    def _zrow(r, _):
        den_stage[r, :] = jnp.zeros((16,), jnp.float32)
        return 0
    lax.fori_loop(0, 128, _zrow, 0)

    def _zr(r, _):
        zden[r, :] = jnp.zeros((16,), jnp.float32)
        return 0
    lax.fori_loop(0, 400, _zr, 0)
    for q in range(8):
        ch = s + q * NS

        @pl.when(ch < 125)
        def _():
            pltpu.sync_copy(zden, shared_den.at[pl.ds(ch * 400, 400)])
    plsc.subcore_barrier()

    nsb = jnp.where(widx < N_BIG1, SB1_BIG, SB1_SMALL)
    sb0 = jnp.where(widx < N_BIG1, widx * SB1_BIG,
                    N_BIG1 * SB1_BIG + (widx - N_BIG1) * SB1_SMALL)

    row_off = iota * F

    NBATCH = SB1 // L

    def _issue(b, slot):
        src16 = src_stage[pl.ds(b * L, L)]
        dst16 = dst_stage[pl.ds(b * L, L)]
        pltpu.async_copy(fsrc_hbm.at[src16],
                         fs_buf.at[pl.ds(slot * L, L)], sem)
        pltpu.async_copy(fdst_hbm.at[dst16],
                         fd_buf.at[pl.ds(slot * L, L)], sem)

    def _drain(slot):
        pltpu.make_async_copy(fsrc_hbm.at[pl.ds(0, L)],
                              fs_buf.at[pl.ds(slot * L, L)], sem).wait()
        pltpu.make_async_copy(fdst_hbm.at[pl.ds(0, L)],
                              fd_buf.at[pl.ds(slot * L, L)], sem).wait()

    def _subblock(k, _):
        base = (sb0 + k) * SB1
        pltpu.sync_copy(src_hbm.at[pl.ds(base, SB1)], src_stage)
        pltpu.sync_copy(dst_hbm.at[pl.ds(base, SB1)], dst_stage)
        _issue(jnp.int32(0), jnp.int32(0))

        def _batch(b, _):
            b2 = lax.rem(b, 8)
            slot = lax.rem(b, 2)
            _drain(slot)

            @pl.when(b + 1 < NBATCH)
            def _():
                _issue(b + 1, 1 - slot)
            src16 = src_stage[pl.ds(b * L, L)]
            dst16 = dst_stage[pl.ds(b * L, L)]
            rows16 = slot * L + iota
            accs = []
            for h in range(HEADS):
                acc = jnp.zeros((L,), jnp.float32)
                for jj in range(HID // L):
                    av = attn_v[h, pl.ds(jj * L, L)]
                    base_d = h * HID + jj * L
                    for k in range(L):
                        rot = (iota + k) % L
                        cols = base_d + rot
                        a = plsc.load_gather(fs_buf, [rows16, cols])
                        bdd = plsc.load_gather(fd_buf, [rows16, cols])
                        z = a + bdd
                        z = jnp.where(z >= 0.0, z, 0.2 * z)
                        acc = acc + av[rot] * z
                accs.append(acc)
            rows = b * L + iota
            drows = b2 * L + iota
            for h in range(HEADS):
                ex = jnp.exp(jnp.clip(accs[h], -60.0, 60.0))
                plsc.store_scatter(rec_stage,
                                   [rows, jnp.full((L,), h, jnp.int32)], ex)
                plsc.store_scatter(den_stage,
                                   [drows, jnp.full((L,), h, jnp.int32)], ex)
            idx_buf[0, pl.ds(b2 * L, L)] = dst16

            @pl.when(b2 == 7)
            def _():
                pltpu.sync_copy(den_stage, shared_den.at[idx_buf.at[0]],
                                add=True)
            return 0
        lax.fori_loop(0, NBATCH, _batch, 0)
        pltpu.sync_copy(rec_stage, rec_hbm.at[pl.ds(base, SB1)])
        return 0

    lax.fori_loop(0, nsb, _subblock, 0)

    plsc.subcore_barrier()
    for q in range(8):
        ch = s + q * NS

        @pl.when(ch < 125)
        def _():
            pltpu.sync_copy(shared_den.at[pl.ds(ch * 400, 400)],
                            den_hbm.at[c, pl.ds(ch * 400, 400)])


def _pass1(f_src, f_dst, src_ids, dst_ids, attn):
    mesh = plsc.VectorSubcoreMesh(core_axis_name="c", subcore_axis_name="s")
    kfn = pl.kernel(
        _p1_body,
        compiler_params=pltpu.CompilerParams(use_tc_tiling_on_sc=False, needs_layout_passes=False),
        out_type=(jax.ShapeDtypeStruct((E, 8), jnp.float32),
                  jax.ShapeDtypeStruct((2, NE, 16), jnp.float32)),
        mesh=mesh,
        scratch_types=[
            pltpu.VMEM((HEADS, HID), jnp.float32),
            pltpu.VMEM((SB1,), jnp.int32),
            pltpu.VMEM((SB1,), jnp.int32),
            pltpu.VMEM((2 * L, F), jnp.float32),
            pltpu.VMEM((2 * L, F), jnp.float32),
            pltpu.VMEM((SB1, 8), jnp.float32),
            pltpu.VMEM((128, 16), jnp.float32),
            pltpu.VMEM((1, 128), jnp.int32),
            pltpu.VMEM((400, 16), jnp.float32),
            pltpu.VMEM_SHARED((NE, 16), jnp.float32),
            pltpu.SemaphoreType.DMA,
        ],
    )
    return kfn(f_src, f_dst, src_ids, dst_ids, attn)


def _p2_flush(n, fcnt, rec_hbm, fsrc_hbm, eid_buf, drel_buf, scomp_buf,
              gid_buf, drl_buf, src_buf, rec_v, contrib, shared_num,
              sem, sem2, iota):
    fslot = lax.rem(fcnt, 2)

    @pl.when(fcnt >= 2)
    def _():
        pltpu.make_async_copy(contrib.at[fslot],
                              shared_num.at[drl_buf.at[fslot]], sem2).wait()
    for sb in range(FL // L):
        m = sb * L + iota < n
        ev = eid_buf[pl.ds(sb * L, L)]
        gid_buf[0, pl.ds(sb * L, L)] = jnp.where(m, ev, 0)
        dv = drel_buf[pl.ds(sb * L, L)]
        drl_buf[fslot, pl.ds(sb * L, L)] = jnp.where(m, dv, 0)
        sv = scomp_buf[pl.ds(sb * L, L)]
        src_buf[0, pl.ds(sb * L, L)] = jnp.where(m, sv, 0)
    cp1 = pltpu.async_copy(rec_hbm.at[gid_buf.at[0]], rec_v, sem)
    cp2 = pltpu.async_copy(fsrc_hbm.at[src_buf.at[0]], contrib.at[fslot],
                           sem)
    cp1.wait()
    cp2.wait()

    def _group16(g, _):
        rows = g * L + iota
        m = rows < n
        ex_vs = [jnp.where(m, plsc.load_gather(
            rec_v, [rows, jnp.full((L,), h, jnp.int32)]), 0.0)
            for h in range(HEADS)]
        for r2 in range(L):
            r = g * L + r2
            for h in range(HEADS):
                w = ex_vs[h][r2]
                for j in range(HID // L):
                    d0 = h * HID + j * L
                    contrib[fslot, r, pl.ds(d0, L)] = \
                        w * contrib[fslot, r, pl.ds(d0, L)]
        return 0
    lax.fori_loop(0, FL // L, _group16, 0)
    pltpu.async_copy(contrib.at[fslot], shared_num.at[drl_buf.at[fslot]],
                     sem2, add=True)


def _p2_body(rec_hbm, fsrc_hbm, src_hbm, dst_hbm, num_hbm,
             dst_stage, src_stage, eid_buf, drel_buf, scomp_buf, gid_buf,
             drl_buf, src_buf, rec_v, contrib, zbuf, shared_num,
             sem, sem2):
    c = lax.axis_index("c")
    s = lax.axis_index("s")
    iota = jnp.arange(L, dtype=jnp.int32)

    def _zr(r, _):
        for j in range(F // L):
            zbuf[r, pl.ds(j * L, L)] = jnp.zeros((L,), jnp.float32)
        return 0
    lax.fori_loop(0, 16, _zr, 0)

    e0 = s * PER_SUB2
    r0 = s * 128

    def _chunk(i, _):
        chunk = 2 * i + c
        lo = chunk * CH

        @pl.when(chunk < NCH)
        def _chunk_body():
            _chunk_inner(chunk, lo)
        return 0

    def _chunk_inner(chunk, lo):
        @pl.when(s < NS - 1)
        def _():
            cps = [pltpu.async_copy(
                zbuf, shared_num.at[pl.ds(r0 + 16 * q, 16)], sem)
                for q in range(8)]
            for cp in cps:
                cp.wait()

        @pl.when(s == NS - 1)
        def _():
            cps = [pltpu.async_copy(
                zbuf, shared_num.at[pl.ds(1920 + 16 * q, 16)], sem)
                for q in range(5)]
            for cp in cps:
                cp.wait()
        plsc.subcore_barrier()

        def _sissue(k, slot):
            base = e0 + k * SB2
            pltpu.async_copy(dst_hbm.at[pl.ds(base, SB2)],
                             dst_stage.at[slot], sem)
            pltpu.async_copy(src_hbm.at[pl.ds(base, SB2)],
                             src_stage.at[slot], sem)

        def _sdrain(slot):
            pltpu.make_async_copy(dst_hbm.at[pl.ds(0, SB2)],
                                  dst_stage.at[slot], sem).wait()
            pltpu.make_async_copy(src_hbm.at[pl.ds(0, SB2)],
                                  src_stage.at[slot], sem).wait()

        _sissue(jnp.int32(0), jnp.int32(0))

        def _scan_sub(k, carry):
            base = e0 + k * SB2
            slot = lax.rem(k, 2)
            _sdrain(slot)

            @pl.when(k + 1 < NSB2)
            def _():
                _sissue(k + 1, 1 - slot)

            def _scan_batch(b, carry):
                cnt, fcnt = carry
                d16 = dst_stage[slot, pl.ds(b * L, L)]
                s16 = src_stage[slot, pl.ds(b * L, L)]
                drel = d16 - lo
                m = (drel >= 0) & (drel < CH)
                eidv = base + b * L + iota
                plsc.store_compressed(eid_buf.at[pl.ds(cnt, L)], eidv,
                                      mask=m)
                plsc.store_compressed(drel_buf.at[pl.ds(cnt, L)], drel,
                                      mask=m)
                plsc.store_compressed(scomp_buf.at[pl.ds(cnt, L)], s16,
                                      mask=m)
                cnt = cnt + jnp.sum(m.astype(jnp.int32))

                def _do_flush(carry):
                    cnt, fcnt = carry
                    _p2_flush(jnp.int32(FL), fcnt, rec_hbm, fsrc_hbm,
                              eid_buf, drel_buf, scomp_buf, gid_buf, drl_buf,
                              src_buf, rec_v, contrib, shared_num,
                              sem, sem2, iota)
                    eid_buf[pl.ds(0, L)] = eid_buf[pl.ds(FL, L)]
                    drel_buf[pl.ds(0, L)] = drel_buf[pl.ds(FL, L)]
                    scomp_buf[pl.ds(0, L)] = scomp_buf[pl.ds(FL, L)]
                    return (cnt - FL, fcnt + 1)

                return lax.cond(cnt >= FL, _do_flush, lambda x: x,
                                (cnt, fcnt))

            return lax.fori_loop(0, SB2 // L, _scan_batch, carry)

        cnt, fcnt = lax.fori_loop(0, NSB2, _scan_sub,
                                  (jnp.int32(0), jnp.int32(0)))

        @pl.when(cnt > 0)
        def _():
            _p2_flush(cnt, fcnt, rec_hbm, fsrc_hbm, eid_buf, drel_buf,
                      scomp_buf, gid_buf, drl_buf, src_buf, rec_v,
                      contrib, shared_num, sem, sem2, iota)
        fcnt = fcnt + jnp.where(cnt > 0, 1, 0)

        @pl.when(fcnt >= 1)
        def _():
            pltpu.make_async_copy(
                contrib.at[lax.rem(fcnt - 1, 2)],
                shared_num.at[drl_buf.at[lax.rem(fcnt - 1, 2)]], sem2).wait()

        @pl.when(fcnt >= 2)
        def _():
            pltpu.make_async_copy(
                contrib.at[lax.rem(fcnt, 2)],
                shared_num.at[drl_buf.at[lax.rem(fcnt, 2)]], sem2).wait()

        plsc.subcore_barrier()

        @pl.when(s < NS - 1)
        def _():
            pltpu.sync_copy(shared_num.at[pl.ds(r0, 128)],
                            num_hbm.at[pl.ds(lo + r0, 128)])

        @pl.when(s == NS - 1)
        def _():
            pltpu.sync_copy(shared_num.at[pl.ds(1920, 80)],
                            num_hbm.at[pl.ds(lo + 1920, 80)])

    lax.fori_loop(0, (NCH + 1) // 2, _chunk, 0)


def _pass2(rec, f_src, src_ids, dst_ids):
    mesh = plsc.VectorSubcoreMesh(core_axis_name="c", subcore_axis_name="s")
    kfn = pl.kernel(
        _p2_body,
        compiler_params=pltpu.CompilerParams(use_tc_tiling_on_sc=False, needs_layout_passes=False),
        out_type=jax.ShapeDtypeStruct((NE, F), jnp.float32),
        mesh=mesh,
        scratch_types=[
            pltpu.VMEM((2, SB2), jnp.int32),
            pltpu.VMEM((2, SB2), jnp.int32),
            pltpu.VMEM((FL + 32,), jnp.int32),
            pltpu.VMEM((FL + 32,), jnp.int32),
            pltpu.VMEM((FL + 32,), jnp.int32),
            pltpu.VMEM((1, FL), jnp.int32),
            pltpu.VMEM((2, FL), jnp.int32),
            pltpu.VMEM((1, FL), jnp.int32),
            pltpu.VMEM((FL, 8), jnp.float32),
            pltpu.VMEM((2, FL, F), jnp.float32),
            pltpu.VMEM((16, F), jnp.float32),
            pltpu.VMEM_SHARED((CH, F), jnp.float32),
            pltpu.SemaphoreType.DMA,
            pltpu.SemaphoreType.DMA,
        ],
    )
    return kfn(rec, f_src, src_ids, dst_ids)


def kernel(vehicle_features, edge_node_features, edge_index,
           w_ve, b_ve, g_ve, be_ve,
           w_ee, b_ee, g_ee, be_ee,
           w_src, b_src, w_dst, b_dst, attn, conv_bias,
           w_fp, b_fp):
    f_src = _encode(vehicle_features, w_ve, b_ve, g_ve, be_ve, w_src, b_src)
    f_dst = _encode(edge_node_features, w_ee, b_ee, g_ee, be_ee, w_dst, b_dst)
    src_ids = edge_index[0]
    dst_ids = edge_index[1]
    rec, den2 = _pass1(f_src, f_dst, src_ids, dst_ids, attn)
    num = _pass2(rec, f_src, src_ids, dst_ids)
    return _finalize(num, den2, conv_bias, w_fp, b_fp)


# speedup vs baseline: 1.2912x; 1.2908x over previous
import functools

import jax
import jax.numpy as jnp
from jax import lax
from jax.experimental import pallas as pl
from jax.experimental.pallas import tpu as pltpu
from jax.experimental.pallas import tpu_sc as plsc

NV = 50000
NE = 50000
E = 800000
VDIM = 128
HID = 64
HEADS = 4
F = HEADS * HID

L = 16
NC = 2
NS = 16
NW = NC * NS

SB1 = 640
NSB1 = E // SB1
SB1_BIG = NSB1 // NW + 1
SB1_SMALL = NSB1 // NW
N_BIG1 = NSB1 - SB1_SMALL * NW

NCH = 25
CH = NE // NCH
SB2 = 2000
PER_SUB2 = E // NS
NSB2 = PER_SUB2 // SB2
FL = 128

_iota16 = None


def _enc_body(x_ref, w1_ref, b1_ref, g_ref, be_ref, w2_ref, b2_ref, o_ref,
              obf_ref=None):
    x = x_ref[...]
    h = jnp.maximum(jnp.dot(x, w1_ref[...], preferred_element_type=jnp.float32)
                    + b1_ref[...][None, :], 0.0)
    mu = jnp.mean(h, axis=1, keepdims=True)
    var = jnp.mean((h - mu) * (h - mu), axis=1, keepdims=True)
    y = (h - mu) * lax.rsqrt(var + 1e-5) * g_ref[...][None, :] + be_ref[...][None, :]
    f = jnp.dot(y, w2_ref[...], preferred_element_type=jnp.float32) \
        + b2_ref[...][None, :]
    o_ref[...] = f
    if obf_ref is not None:
        obf_ref[...] = f.astype(jnp.bfloat16)


def _encode(x, w1, b1, g, be, w2, b2, want_bf16=False):
    B = 2000
    n = x.shape[0]
    grid = n // B
    if want_bf16:
        out_specs = (pl.BlockSpec((B, F), lambda i: (i, 0)),
                     pl.BlockSpec((B, F), lambda i: (i, 0)))
        out_shape = (jax.ShapeDtypeStruct((n, F), jnp.float32),
                     jax.ShapeDtypeStruct((n, F), jnp.bfloat16))
        body = _enc_body
    else:
        out_specs = pl.BlockSpec((B, F), lambda i: (i, 0))
        out_shape = jax.ShapeDtypeStruct((n, F), jnp.float32)

        def body(x_ref, w1_ref, b1_ref, g_ref, be_ref, w2_ref, b2_ref, o_ref):
            _enc_body(x_ref, w1_ref, b1_ref, g_ref, be_ref, w2_ref, b2_ref,
                      o_ref)
    return pl.pallas_call(
        body,
        grid=(grid,),
        in_specs=[
            pl.BlockSpec((B, VDIM), lambda i: (i, 0)),
            pl.BlockSpec((VDIM, HID), lambda i: (0, 0)),
            pl.BlockSpec((HID,), lambda i: (0,)),
            pl.BlockSpec((HID,), lambda i: (0,)),
            pl.BlockSpec((HID,), lambda i: (0,)),
            pl.BlockSpec((HID, F), lambda i: (0, 0)),
            pl.BlockSpec((F,), lambda i: (0,)),
        ],
        out_specs=out_specs,
        out_shape=out_shape,
    )(x, w1, b1, g, be, w2, b2)


def _fin_body(num_ref, den_ref, cb_ref, wfp_ref, bfp_ref, o_ref):
    den = den_ref[0] + den_ref[1]
    num = num_ref[...]
    acc = jnp.zeros((num.shape[0], HID), jnp.float32)
    for h in range(HEADS):
        dh = den[:, h:h + 1]
        dh = jnp.where(dh > 0.0, dh, 1.0)
        acc = acc + num[:, h * HID:(h + 1) * HID] / dh \
            + cb_ref[...][None, h * HID:(h + 1) * HID]
    eo = acc * (1.0 / HEADS)
    o_ref[...] = jnp.maximum(
        jnp.dot(eo, wfp_ref[...], preferred_element_type=jnp.float32)
        + bfp_ref[...][None, :], 0.0)


def _finalize(num, den2, conv_bias, w_fp, b_fp):
    B = 2000
    grid = NE // B
    return pl.pallas_call(
        _fin_body,
        grid=(grid,),
        in_specs=[
            pl.BlockSpec((B, F), lambda i: (i, 0)),
            pl.BlockSpec((2, B, 16), lambda i: (0, i, 0)),
            pl.BlockSpec((F,), lambda i: (0,)),
            pl.BlockSpec((HID, HID), lambda i: (0, 0)),
            pl.BlockSpec((HID,), lambda i: (0,)),
        ],
        out_specs=pl.BlockSpec((B, HID), lambda i: (i, 0)),
        out_shape=jax.ShapeDtypeStruct((NE, HID), jnp.float32),
    )(num, den2, conv_bias, w_fp, b_fp)


def _p1_body(fsrc_hbm, fdst_hbm, src_hbm, dst_hbm, attn_hbm, rec_hbm, den_hbm,
             attn_v, src_stage, dst_stage, fs_buf, fd_buf, rec_stage,
             den_stage, idx_buf, zden, shared_den, sem):
    c = lax.axis_index("c")
    s = lax.axis_index("s")
    widx = s * NC + c
    iota = jnp.arange(L, dtype=jnp.int32)

    pltpu.sync_copy(attn_hbm, attn_v)

    def _zrow(r, _):
        den_stage[r, :] = jnp.zeros((16,), jnp.float32)
        return 0
    lax.fori_loop(0, 128, _zrow, 0)

    def _zr(r, _):
        zden[r, :] = jnp.zeros((16,), jnp.float32)
        return 0
    lax.fori_loop(0, 400, _zr, 0)
    for q in range(8):
        ch = s + q * NS

        @pl.when(ch < 125)
        def _():
            pltpu.sync_copy(zden, shared_den.at[pl.ds(ch * 400, 400)])
    plsc.subcore_barrier()

    nsb = jnp.where(widx < N_BIG1, SB1_BIG, SB1_SMALL)
    sb0 = jnp.where(widx < N_BIG1, widx * SB1_BIG,
                    N_BIG1 * SB1_BIG + (widx - N_BIG1) * SB1_SMALL)

    row_off = iota * F

    NBATCH = SB1 // L

    def _issue(b, slot):
        src16 = src_stage[pl.ds(b * L, L)]
        dst16 = dst_stage[pl.ds(b * L, L)]
        pltpu.async_copy(fsrc_hbm.at[src16],
                         fs_buf.at[pl.ds(slot * L, L)], sem)
        pltpu.async_copy(fdst_hbm.at[dst16],
                         fd_buf.at[pl.ds(slot * L, L)], sem)

    def _drain(slot):
        pltpu.make_async_copy(fsrc_hbm.at[pl.ds(0, L)],
                              fs_buf.at[pl.ds(slot * L, L)], sem).wait()
        pltpu.make_async_copy(fdst_hbm.at[pl.ds(0, L)],
                              fd_buf.at[pl.ds(slot * L, L)], sem).wait()

    def _subblock(k, _):
        base = (sb0 + k) * SB1
        pltpu.sync_copy(src_hbm.at[pl.ds(base, SB1)], src_stage)
        pltpu.sync_copy(dst_hbm.at[pl.ds(base, SB1)], dst_stage)
        _issue(jnp.int32(0), jnp.int32(0))

        def _batch(b, _):
            b2 = lax.rem(b, 8)
            slot = lax.rem(b, 2)
            _drain(slot)

            @pl.when(b + 1 < NBATCH)
            def _():
                _issue(b + 1, 1 - slot)
            src16 = src_stage[pl.ds(b * L, L)]
            dst16 = dst_stage[pl.ds(b * L, L)]
            rows16 = slot * L + iota
            accs = []
            for h in range(HEADS):
                acc = jnp.zeros((L,), jnp.float32)
                for jj in range(HID // L):
                    av = attn_v[h, pl.ds(jj * L, L)]
                    base_d = h * HID + jj * L
                    for k in range(L):
                        rot = (iota + k) % L
                        cols = base_d + rot
                        a = plsc.load_gather(fs_buf, [rows16, cols])
                        bdd = plsc.load_gather(fd_buf, [rows16, cols])
                        z = a + bdd
                        z = jnp.where(z >= 0.0, z, 0.2 * z)
                        acc = acc + av[rot] * z
                accs.append(acc)
            rows = b * L + iota
            drows = b2 * L + iota
            for h in range(HEADS):
                ex = jnp.exp(jnp.clip(accs[h], -60.0, 60.0))
                plsc.store_scatter(rec_stage,
                                   [rows, jnp.full((L,), h, jnp.int32)], ex)
                plsc.store_scatter(den_stage,
                                   [drows, jnp.full((L,), h, jnp.int32)], ex)
            idx_buf[0, pl.ds(b2 * L, L)] = dst16

            @pl.when(b2 == 7)
            def _():
                pltpu.sync_copy(den_stage, shared_den.at[idx_buf.at[0]],
                                add=True)
            return 0
        lax.fori_loop(0, NBATCH, _batch, 0)
        pltpu.sync_copy(rec_stage, rec_hbm.at[pl.ds(base, SB1)])
        return 0

    lax.fori_loop(0, nsb, _subblock, 0)

    plsc.subcore_barrier()
    for q in range(8):
        ch = s + q * NS

        @pl.when(ch < 125)
        def _():
            pltpu.sync_copy(shared_den.at[pl.ds(ch * 400, 400)],
                            den_hbm.at[c, pl.ds(ch * 400, 400)])


def _pass1(f_src, f_dst, src_ids, dst_ids, attn):
    mesh = plsc.VectorSubcoreMesh(core_axis_name="c", subcore_axis_name="s")
    kfn = pl.kernel(
        _p1_body,
        compiler_params=pltpu.CompilerParams(use_tc_tiling_on_sc=False, needs_layout_passes=False),
        out_type=(jax.ShapeDtypeStruct((E, 8), jnp.float32),
                  jax.ShapeDtypeStruct((2, NE, 16), jnp.float32)),
        mesh=mesh,
        scratch_types=[
            pltpu.VMEM((HEADS, HID), jnp.float32),
            pltpu.VMEM((SB1,), jnp.int32),
            pltpu.VMEM((SB1,), jnp.int32),
            pltpu.VMEM((2 * L, F), jnp.float32),
            pltpu.VMEM((2 * L, F), jnp.float32),
            pltpu.VMEM((SB1, 8), jnp.float32),
            pltpu.VMEM((128, 16), jnp.float32),
            pltpu.VMEM((1, 128), jnp.int32),
            pltpu.VMEM((400, 16), jnp.float32),
            pltpu.VMEM_SHARED((NE, 16), jnp.float32),
            pltpu.SemaphoreType.DMA,
        ],
    )
    return kfn(f_src, f_dst, src_ids, dst_ids, attn)


def _p2_flush(n, rec_hbm, fsrc_hbm, eid_buf, drel_buf, scomp_buf,
              gid_buf, drl_buf, src_buf, rec_v, fs_v, contrib, shared_num,
              sem, iota):
    for sb in range(FL // L):
        m = sb * L + iota < n
        ev = eid_buf[pl.ds(sb * L, L)]
        gid_buf[0, pl.ds(sb * L, L)] = jnp.where(m, ev, 0)
        dv = drel_buf[pl.ds(sb * L, L)]
        drl_buf[0, pl.ds(sb * L, L)] = jnp.where(m, dv, 0)
        sv = scomp_buf[pl.ds(sb * L, L)]
        src_buf[0, pl.ds(sb * L, L)] = jnp.where(m, sv, 0)
    cp1 = pltpu.async_copy(rec_hbm.at[gid_buf.at[0]], rec_v, sem)
    cp2 = pltpu.async_copy(fsrc_hbm.at[src_buf.at[0]], fs_v, sem)
    cp1.wait()
    cp2.wait()

    def _group16(g, _):
        rows = g * L + iota
        m = rows < n
        ex_vs = [jnp.where(m, plsc.load_gather(
            rec_v, [rows, jnp.full((L,), h, jnp.int32)]), 0.0)
            for h in range(HEADS)]
        for r2 in range(L):
            r = g * L + r2
            for h in range(HEADS):
                w = ex_vs[h][r2]
                for j in range(HID // L):
                    d0 = h * HID + j * L
                    contrib[r, pl.ds(d0, L)] = w * fs_v[r, pl.ds(d0, L)]
        return 0
    lax.fori_loop(0, FL // L, _group16, 0)
    pltpu.sync_copy(contrib, shared_num.at[drl_buf.at[0]], add=True)


def _p2_body(rec_hbm, fsrc_hbm, src_hbm, dst_hbm, num_hbm,
             dst_stage, src_stage, eid_buf, drel_buf, scomp_buf, gid_buf,
             drl_buf, src_buf, rec_v, fs_v, contrib, zbuf, shared_num,
             sem):
    c = lax.axis_index("c")
    s = lax.axis_index("s")
    iota = jnp.arange(L, dtype=jnp.int32)

    def _zr(r, _):
        for j in range(F // L):
            zbuf[r, pl.ds(j * L, L)] = jnp.zeros((L,), jnp.float32)
        return 0
    lax.fori_loop(0, 16, _zr, 0)

    e0 = s * PER_SUB2
    r0 = s * 128

    def _chunk(i, _):
        chunk = 2 * i + c
        lo = chunk * CH

        @pl.when(chunk < NCH)
        def _chunk_body():
            _chunk_inner(chunk, lo)
        return 0

    def _chunk_inner(chunk, lo):
        @pl.when(s < NS - 1)
        def _():
            cps = [pltpu.async_copy(
                zbuf, shared_num.at[pl.ds(r0 + 16 * q, 16)], sem)
                for q in range(8)]
            for cp in cps:
                cp.wait()

        @pl.when(s == NS - 1)
        def _():
            cps = [pltpu.async_copy(
                zbuf, shared_num.at[pl.ds(1920 + 16 * q, 16)], sem)
                for q in range(5)]
            for cp in cps:
                cp.wait()
        plsc.subcore_barrier()

        def _sissue(k, slot):
            base = e0 + k * SB2
            pltpu.async_copy(dst_hbm.at[pl.ds(base, SB2)],
                             dst_stage.at[slot], sem)
            pltpu.async_copy(src_hbm.at[pl.ds(base, SB2)],
                             src_stage.at[slot], sem)

        def _sdrain(slot):
            pltpu.make_async_copy(dst_hbm.at[pl.ds(0, SB2)],
                                  dst_stage.at[slot], sem).wait()
            pltpu.make_async_copy(src_hbm.at[pl.ds(0, SB2)],
                                  src_stage.at[slot], sem).wait()

        _sissue(jnp.int32(0), jnp.int32(0))

        def _scan_sub(k, carry):
            base = e0 + k * SB2
            slot = lax.rem(k, 2)
            _sdrain(slot)

            @pl.when(k + 1 < NSB2)
            def _():
                _sissue(k + 1, 1 - slot)

            def _scan_batch(b, cnt):
                d16 = dst_stage[slot, pl.ds(b * L, L)]
                s16 = src_stage[slot, pl.ds(b * L, L)]
                drel = d16 - lo
                m = (drel >= 0) & (drel < CH)
                eidv = base + b * L + iota
                plsc.store_compressed(eid_buf.at[pl.ds(cnt, L)], eidv,
                                      mask=m)
                plsc.store_compressed(drel_buf.at[pl.ds(cnt, L)], drel,
                                      mask=m)
                plsc.store_compressed(scomp_buf.at[pl.ds(cnt, L)], s16,
                                      mask=m)
                cnt = cnt + jnp.sum(m.astype(jnp.int32))

                def _do_flush(cnt):
                    _p2_flush(jnp.int32(FL), rec_hbm, fsrc_hbm,
                              eid_buf, drel_buf, scomp_buf, gid_buf, drl_buf,
                              src_buf, rec_v, fs_v, contrib, shared_num,
                              sem, iota)
                    eid_buf[pl.ds(0, L)] = eid_buf[pl.ds(FL, L)]
                    drel_buf[pl.ds(0, L)] = drel_buf[pl.ds(FL, L)]
                    scomp_buf[pl.ds(0, L)] = scomp_buf[pl.ds(FL, L)]
                    return cnt - FL

                return lax.cond(cnt >= FL, _do_flush, lambda x: x, cnt)

            return lax.fori_loop(0, SB2 // L, _scan_batch, carry)

        cnt = lax.fori_loop(0, NSB2, _scan_sub, jnp.int32(0))

        @pl.when(cnt > 0)
        def _():
            _p2_flush(cnt, rec_hbm, fsrc_hbm, eid_buf, drel_buf,
                      scomp_buf, gid_buf, drl_buf, src_buf, rec_v, fs_v,
                      contrib, shared_num, sem, iota)

        plsc.subcore_barrier()

        @pl.when(s < NS - 1)
        def _():
            pltpu.sync_copy(shared_num.at[pl.ds(r0, 128)],
                            num_hbm.at[pl.ds(lo + r0, 128)])

        @pl.when(s == NS - 1)
        def _():
            pltpu.sync_copy(shared_num.at[pl.ds(1920, 80)],
                            num_hbm.at[pl.ds(lo + 1920, 80)])

    lax.fori_loop(0, (NCH + 1) // 2, _chunk, 0)


def _pass2(rec, f_src, src_ids, dst_ids):
    mesh = plsc.VectorSubcoreMesh(core_axis_name="c", subcore_axis_name="s")
    kfn = pl.kernel(
        _p2_body,
        compiler_params=pltpu.CompilerParams(use_tc_tiling_on_sc=False, needs_layout_passes=False),
        out_type=jax.ShapeDtypeStruct((NE, F), jnp.float32),
        mesh=mesh,
        scratch_types=[
            pltpu.VMEM((2, SB2), jnp.int32),
            pltpu.VMEM((2, SB2), jnp.int32),
            pltpu.VMEM((FL + 32,), jnp.int32),
            pltpu.VMEM((FL + 32,), jnp.int32),
            pltpu.VMEM((FL + 32,), jnp.int32),
            pltpu.VMEM((1, FL), jnp.int32),
            pltpu.VMEM((1, FL), jnp.int32),
            pltpu.VMEM((1, FL), jnp.int32),
            pltpu.VMEM((FL, 8), jnp.float32),
            pltpu.VMEM((FL, F), jnp.float32),
            pltpu.VMEM((FL, F), jnp.float32),
            pltpu.VMEM((16, F), jnp.float32),
            pltpu.VMEM_SHARED((CH, F), jnp.float32),
            pltpu.SemaphoreType.DMA,
        ],
    )
    return kfn(rec, f_src, src_ids, dst_ids)


def kernel(vehicle_features, edge_node_features, edge_index,
           w_ve, b_ve, g_ve, be_ve,
           w_ee, b_ee, g_ee, be_ee,
           w_src, b_src, w_dst, b_dst, attn, conv_bias,
           w_fp, b_fp):
    f_src = _encode(vehicle_features, w_ve, b_ve, g_ve, be_ve, w_src, b_src)
    f_dst = _encode(edge_node_features, w_ee, b_ee, g_ee, be_ee, w_dst, b_dst)
    src_ids = edge_index[0]
    dst_ids = edge_index[1]
    rec, den2 = _pass1(f_src, f_dst, src_ids, dst_ids, attn)
    num = _pass2(rec, f_src, src_ids, dst_ids)
    return _finalize(num, den2, conv_bias, w_fp, b_fp)
